# baseline (speedup 1.0000x reference)
import functools

import jax
import jax.numpy as jnp
from jax import lax
from jax.experimental import pallas as pl
from jax.experimental.pallas import tpu as pltpu
from jax.experimental.pallas import tpu_sc as plsc

B = 4096
D = 64
DW = 80
NCAT = 26
NNUM = 13
S = 40
WPAD = 16
NCORE = 2
NSUB = 16
NW = NCORE * NSUB
RW = B // NW
RPG = 2
NBUF = 8
NG = RW // RPG
LANE = 16
CD = D // LANE
CB = RW // LANE


def _sc_pool(idx_flat, w_flat, aug_table):
    mesh = plsc.VectorSubcoreMesh(
        core_axis_name="c", subcore_axis_name="s",
        num_cores=NCORE, num_subcores=NSUB)

    @functools.partial(
        pl.kernel,
        out_type=(jax.ShapeDtypeStruct((B, D), jnp.float32),
                  jax.ShapeDtypeStruct((B,), jnp.float32)),
        mesh=mesh,
        scratch_types=[
            pltpu.VMEM((RW * S,), jnp.int32),
            pltpu.VMEM((RW * WPAD,), jnp.float32),
            pltpu.VMEM((NBUF, RPG * S, DW), jnp.float32),
            pltpu.VMEM((RW, D), jnp.float32),
            pltpu.VMEM((RW * LANE,), jnp.float32),
            pltpu.VMEM((RW,), jnp.float32),
        ] + [pltpu.SemaphoreType.DMA for _ in range(NBUF)],
        compiler_params=pltpu.CompilerParams(
            use_tc_tiling_on_sc=False, needs_layout_passes=False),
    )
    def k(idx_f_h, w_f_h, aug_h, so_h, fo_h,
          idx_v, w_v, ebuf, so_v, fo_lanes, fo_v, *esems):
        wid = lax.axis_index("s") * NCORE + lax.axis_index("c")
        base = wid * RW

        pltpu.sync_copy(idx_f_h.at[pl.ds(base * S, RW * S)], idx_v)
        pltpu.sync_copy(w_f_h.at[pl.ds(base * WPAD, RW * WPAD)], w_v)

        for g in range(NBUF):
            pltpu.async_copy(
                aug_h.at[idx_v.at[pl.ds(g * RPG * S, RPG * S)]],
                ebuf.at[g], esems[g])

        zi = jnp.zeros((LANE,), jnp.int32)

        def ring_body(o, carry):
            for slot in range(NBUF):
                g = o * NBUF + slot
                pltpu.make_async_copy(
                    aug_h.at[idx_v.at[pl.ds(g * RPG * S, RPG * S)]],
                    ebuf.at[slot], esems[slot]).wait()
                for rr in range(RPG):
                    i = g * RPG + rr
                    r0 = rr * S
                    s = [jnp.zeros((LANE,), jnp.float32) for _ in range(CD)]
                    ss = [jnp.zeros((LANE,), jnp.float32) for _ in range(CD)]
                    fo_acc = jnp.zeros((LANE,), jnp.float32)
                    for j in range(NCAT):
                        for c in range(CD):
                            v = ebuf[slot, r0 + j, pl.ds(c * LANE, LANE)]
                            s[c] = s[c] + v
                            ss[c] = ss[c] + v * v
                        fo_acc = fo_acc + ebuf[slot, r0 + j, pl.ds(D, LANE)]
                    for t in range(NNUM):
                        wb = plsc.load_gather(w_v, [zi + (i * WPAD + t)])
                        for c in range(CD):
                            v = ebuf[slot, r0 + NCAT + t,
                                     pl.ds(c * LANE, LANE)] * wb
                            s[c] = s[c] + v
                            ss[c] = ss[c] + v * v
                        fo_acc = fo_acc + ebuf[
                            slot, r0 + NCAT + t, pl.ds(D, LANE)] * wb
                    for c in range(CD):
                        so_v[i, pl.ds(c * LANE, LANE)] = (
                            0.5 * (s[c] * s[c] - ss[c]))
                    fo_lanes[pl.ds(i * LANE, LANE)] = fo_acc

                @pl.when(g + NBUF < NG)
                def _():
                    pltpu.async_copy(
                        aug_h.at[idx_v.at[pl.ds((g + NBUF) * RPG * S,
                                                RPG * S)]],
                        ebuf.at[slot], esems[slot])
            return carry
        lax.fori_loop(0, NG // NBUF, ring_body, 0)

        lanes16 = lax.iota(jnp.int32, LANE) * LANE
        for c in range(CB):
            fo_v[pl.ds(c * LANE, LANE)] = plsc.load_gather(
                fo_lanes, [lanes16 + (c * LANE * LANE)])

        pltpu.sync_copy(so_v, so_h.at[pl.ds(base, RW)])
        pltpu.sync_copy(fo_v, fo_h.at[pl.ds(base, RW)])

    return k(idx_flat, w_flat, aug_table)


def _mlp(so, fo, W1, b1, W2, b2, W3t, b3):
    GB = 4
    BB = B // GB

    def body(so_ref, fo_ref, w1_ref, b1_ref, w2_ref, b2_ref, w3_ref, b3_ref,
             out_ref):
        h = jnp.dot(so_ref[...], w1_ref[...],
                    preferred_element_type=jnp.float32)
        h = jnp.maximum(h + b1_ref[...], 0.0)
        h = jnp.dot(h, w2_ref[...], preferred_element_type=jnp.float32)
        h = jnp.maximum(h + b2_ref[...], 0.0)
        z = jnp.sum(h * w3_ref[...], axis=1, keepdims=True) + b3_ref[0, 0]
        out_ref[...] = fo_ref[...] + jax.nn.sigmoid(z)

    return pl.pallas_call(
        body,
        grid=(GB,),
        in_specs=[
            pl.BlockSpec((BB, D), lambda i: (i, 0)),
            pl.BlockSpec((BB, 1), lambda i: (i, 0)),
            pl.BlockSpec((D, 64), lambda i: (0, 0)),
            pl.BlockSpec((1, 64), lambda i: (0, 0)),
            pl.BlockSpec((64, 32), lambda i: (0, 0)),
            pl.BlockSpec((1, 32), lambda i: (0, 0)),
            pl.BlockSpec((1, 32), lambda i: (0, 0)),
            pl.BlockSpec((1, 1), lambda i: (0, 0)),
        ],
        out_specs=pl.BlockSpec((BB, 1), lambda i: (i, 0)),
        out_shape=jax.ShapeDtypeStruct((B, 1), jnp.float32),
    )(so, fo, W1, b1, W2, b2, W3t, b3)


def kernel(category_index, numerical_index, numerical_value, emb_table,
           lin_table, W1, b1, W2, b2, W3, b3):
    F = emb_table.shape[0]
    ci = category_index.astype(jnp.int32)
    ni = numerical_index.astype(jnp.int32)
    nv = numerical_value.astype(jnp.float32)
    idx = jnp.concatenate([ci, ni, jnp.zeros((B, 1), jnp.int32)], axis=1)
    w_flat = jnp.concatenate(
        [nv, jnp.zeros((B, WPAD - NNUM), jnp.float32)], axis=1).reshape(-1)
    aug = jnp.concatenate(
        [emb_table, lin_table,
         jnp.zeros((F, DW - D - 1), jnp.float32)], axis=1)

    so, fo = _sc_pool(idx.reshape(B * S), w_flat, aug)
    out = _mlp(so, fo[:, None], W1, b1.reshape(1, 64), W2, b2.reshape(1, 32),
               W3.T, b3.reshape(1, 1))
    return out


# speedup vs baseline: 1.3993x; 1.3993x over previous
import functools

import jax
import jax.numpy as jnp
import numpy as np
from jax import lax
from jax.experimental import pallas as pl
from jax.experimental.pallas import tpu as pltpu
from jax.experimental.pallas import tpu_sc as plsc

B = 4096
D = 64
NCAT = 26
NNUM = 13
S = 40
WPAD = 16
NCORE = 2
NSUB = 16
NW = NCORE * NSUB
RW = B // NW
RPG = 2
NBUF = 8
NG = RW // RPG
LANE = 16
CB = RW // LANE

_PERM = np.concatenate([
    np.arange(0, 32, 2), np.arange(1, 32, 2),
    np.arange(32, 64, 2), np.arange(33, 64, 2)])


def _sc_pool(idx_flat, w_flat, emb_bf, lin_flat):
    mesh = plsc.VectorSubcoreMesh(
        core_axis_name="c", subcore_axis_name="s",
        num_cores=NCORE, num_subcores=NSUB)

    @functools.partial(
        pl.kernel,
        out_type=(jax.ShapeDtypeStruct((B, D), jnp.float32),
                  jax.ShapeDtypeStruct((B,), jnp.float32)),
        mesh=mesh,
        scratch_types=[
            pltpu.VMEM((RW * S,), jnp.int32),
            pltpu.VMEM((RW * WPAD,), jnp.float32),
            pltpu.VMEM((S, RW), jnp.int32),
            pltpu.VMEM((S, RW), jnp.float32),
            pltpu.VMEM((NBUF, RPG * S, D), jnp.bfloat16),
            pltpu.VMEM((RW, D), jnp.float32),
            pltpu.VMEM((RW,), jnp.float32),
            pltpu.SemaphoreType.DMA,
        ] + [pltpu.SemaphoreType.DMA for _ in range(NBUF)],
        compiler_params=pltpu.CompilerParams(
            use_tc_tiling_on_sc=False, needs_layout_passes=False),
    )
    def k(idx_f_h, w_f_h, emb_h, lin_h, so_h, fo_h,
          idx_v, w_v, idxT_v, lin_v, ebuf, so_v, fo_v, lsem, *esems):
        wid = lax.axis_index("s") * NCORE + lax.axis_index("c")
        base = wid * RW

        pltpu.sync_copy(idx_f_h.at[pl.ds(base * S, RW * S)], idx_v)
        pltpu.sync_copy(w_f_h.at[pl.ds(base * WPAD, RW * WPAD)], w_v)

        for g in range(NBUF):
            pltpu.async_copy(
                emb_h.at[idx_v.at[pl.ds(g * RPG * S, RPG * S)]],
                ebuf.at[g], esems[g])

        zi = jnp.zeros((LANE,), jnp.int32)
        rows16s = lax.iota(jnp.int32, LANE) * S
        rows16w = lax.iota(jnp.int32, LANE) * WPAD

        def tr_body(j, c):
            for gq in range(CB):
                idxT_v[j, pl.ds(gq * LANE, LANE)] = plsc.load_gather(
                    idx_v, [rows16s + (gq * LANE * S + j)])
            return c
        lax.fori_loop(0, S, tr_body, 0)

        def lin_start(j, c):
            pltpu.async_copy(lin_h.at[idxT_v.at[j]], lin_v.at[j], lsem)
            return c
        lax.fori_loop(0, S, lin_start, 0)

        def lin_drain(j, c):
            pltpu.make_async_copy(lin_h.at[idxT_v.at[j]], lin_v.at[j], lsem).wait()
            return c
        lax.fori_loop(0, S, lin_drain, 0)

        for gq in range(CB):
            def cate_acc(j, acc):
                return acc + lin_v[j, pl.ds(gq * LANE, LANE)]
            acc = lax.fori_loop(0, NCAT, cate_acc,
                                jnp.zeros((LANE,), jnp.float32))
            for t in range(NNUM):
                wv = plsc.load_gather(
                    w_v, [rows16w + (gq * LANE * WPAD + t)])
                acc = acc + lin_v[NCAT + t, pl.ds(gq * LANE, LANE)] * wv
            fo_v[pl.ds(gq * LANE, LANE)] = acc

        def ring_body(o, carry):
            for slot in range(NBUF):
                g = o * NBUF + slot
                pltpu.make_async_copy(
                    emb_h.at[idx_v.at[pl.ds(g * RPG * S, RPG * S)]],
                    ebuf.at[slot], esems[slot]).wait()
                for rr in range(RPG):
                    i = g * RPG + rr
                    r0 = rr * S
                    s = [jnp.zeros((LANE,), jnp.float32) for _ in range(4)]
                    ss = [jnp.zeros((LANE,), jnp.float32) for _ in range(4)]
                    for j in range(NCAT):
                        for h in range(2):
                            pair = ebuf[slot, r0 + j, pl.ds(h * 32, 32)]
                            va, vb = plsc.unpack(
                                pair, format=plsc.PackFormat.INTERLEAVED,
                                preferred_element_type=jnp.float32)
                            s[2 * h] = s[2 * h] + va
                            ss[2 * h] = ss[2 * h] + va * va
                            s[2 * h + 1] = s[2 * h + 1] + vb
                            ss[2 * h + 1] = ss[2 * h + 1] + vb * vb
                    for t in range(NNUM):
                        wb = plsc.load_gather(w_v, [zi + (i * WPAD + t)])
                        for h in range(2):
                            pair = ebuf[slot, r0 + NCAT + t,
                                        pl.ds(h * 32, 32)]
                            va, vb = plsc.unpack(
                                pair, format=plsc.PackFormat.INTERLEAVED,
                                preferred_element_type=jnp.float32)
                            va = va * wb
                            vb = vb * wb
                            s[2 * h] = s[2 * h] + va
                            ss[2 * h] = ss[2 * h] + va * va
                            s[2 * h + 1] = s[2 * h + 1] + vb
                            ss[2 * h + 1] = ss[2 * h + 1] + vb * vb
                    for c in range(4):
                        so_v[i, pl.ds(c * LANE, LANE)] = (
                            0.5 * (s[c] * s[c] - ss[c]))

                @pl.when(g + NBUF < NG)
                def _():
                    pltpu.async_copy(
                        emb_h.at[idx_v.at[pl.ds((g + NBUF) * RPG * S,
                                                RPG * S)]],
                        ebuf.at[slot], esems[slot])
            return carry
        lax.fori_loop(0, NG // NBUF, ring_body, 0)

        pltpu.sync_copy(so_v, so_h.at[pl.ds(base, RW)])
        pltpu.sync_copy(fo_v, fo_h.at[pl.ds(base, RW)])

    return k(idx_flat, w_flat, emb_bf, lin_flat)


def _mlp(so, fo, W1, b1, W2, b2, W3t, b3):
    GB = 4
    BB = B // GB

    def body(so_ref, fo_ref, w1_ref, b1_ref, w2_ref, b2_ref, w3_ref, b3_ref,
             out_ref):
        h = jnp.dot(so_ref[...], w1_ref[...],
                    preferred_element_type=jnp.float32)
        h = jnp.maximum(h + b1_ref[...], 0.0)
        h = jnp.dot(h, w2_ref[...], preferred_element_type=jnp.float32)
        h = jnp.maximum(h + b2_ref[...], 0.0)
        z = jnp.sum(h * w3_ref[...], axis=1, keepdims=True) + b3_ref[0, 0]
        out_ref[...] = fo_ref[...] + jax.nn.sigmoid(z)

    return pl.pallas_call(
        body,
        grid=(GB,),
        in_specs=[
            pl.BlockSpec((BB, D), lambda i: (i, 0)),
            pl.BlockSpec((BB, 1), lambda i: (i, 0)),
            pl.BlockSpec((D, 64), lambda i: (0, 0)),
            pl.BlockSpec((1, 64), lambda i: (0, 0)),
            pl.BlockSpec((64, 32), lambda i: (0, 0)),
            pl.BlockSpec((1, 32), lambda i: (0, 0)),
            pl.BlockSpec((1, 32), lambda i: (0, 0)),
            pl.BlockSpec((1, 1), lambda i: (0, 0)),
        ],
        out_specs=pl.BlockSpec((BB, 1), lambda i: (i, 0)),
        out_shape=jax.ShapeDtypeStruct((B, 1), jnp.float32),
    )(so, fo, W1, b1, W2, b2, W3t, b3)


def kernel(category_index, numerical_index, numerical_value, emb_table,
           lin_table, W1, b1, W2, b2, W3, b3):
    ci = category_index.astype(jnp.int32)
    ni = numerical_index.astype(jnp.int32)
    nv = numerical_value.astype(jnp.float32)
    idx = jnp.concatenate([ci, ni, jnp.zeros((B, 1), jnp.int32)], axis=1)
    w_flat = jnp.concatenate(
        [nv, jnp.zeros((B, WPAD - NNUM), jnp.float32)], axis=1).reshape(-1)
    emb_bf = emb_table.astype(jnp.bfloat16)
    lin_flat = lin_table[:, 0]

    so, fo = _sc_pool(idx.reshape(B * S), w_flat, emb_bf, lin_flat)
    out = _mlp(so, fo[:, None], W1[_PERM], b1.reshape(1, 64),
               W2, b2.reshape(1, 32), W3.T, b3.reshape(1, 1))
    return out


# speedup vs baseline: 1.5770x; 1.1269x over previous
import functools

import jax
import jax.numpy as jnp
import numpy as np
from jax import lax
from jax.experimental import pallas as pl
from jax.experimental.pallas import tpu as pltpu
from jax.experimental.pallas import tpu_sc as plsc

B = 4096
D = 64
F = 100000
NCAT = 26
NNUM = 13
S = 40
WPAD = 16
NCORE = 2
NSUB = 16
NW = NCORE * NSUB
RW = B // NW
RPG = 2
NBUF = 4
NG = RW // RPG
LANE = 16
CB = RW // LANE

_PERM = np.concatenate([
    np.arange(0, 32, 2), np.arange(1, 32, 2),
    np.arange(32, 64, 2), np.arange(33, 64, 2)])


def _sc_pool(idx_flat, w_flat, emb_bf, lin_flat):
    mesh = plsc.VectorSubcoreMesh(
        core_axis_name="c", subcore_axis_name="s",
        num_cores=NCORE, num_subcores=NSUB)

    @functools.partial(
        pl.kernel,
        out_type=(jax.ShapeDtypeStruct((B, D), jnp.float32),
                  jax.ShapeDtypeStruct((B,), jnp.float32)),
        mesh=mesh,
        scratch_types=[
            pltpu.VMEM((RW * S,), jnp.int32),
            pltpu.VMEM((RW * WPAD,), jnp.float32),
            pltpu.VMEM((F,), jnp.float32),
            pltpu.VMEM((NBUF, RPG * S, D), jnp.bfloat16),
            pltpu.VMEM((RW, D), jnp.float32),
            pltpu.VMEM((RW,), jnp.float32),
            pltpu.SemaphoreType.DMA,
        ] + [pltpu.SemaphoreType.DMA for _ in range(NBUF)],
        compiler_params=pltpu.CompilerParams(
            use_tc_tiling_on_sc=False, needs_layout_passes=False),
    )
    def k(idx_f_h, w_f_h, emb_h, lin_h, so_h, fo_h,
          idx_v, w_v, lin_t, ebuf, so_v, fo_v, lsem, *esems):
        wid = lax.axis_index("s") * NCORE + lax.axis_index("c")
        base = wid * RW

        pltpu.async_copy(lin_h, lin_t, lsem)

        pltpu.sync_copy(idx_f_h.at[pl.ds(base * S, RW * S)], idx_v)
        pltpu.sync_copy(w_f_h.at[pl.ds(base * WPAD, RW * WPAD)], w_v)

        for g in range(NBUF):
            pltpu.async_copy(
                emb_h.at[idx_v.at[pl.ds(g * RPG * S, RPG * S)]],
                ebuf.at[g], esems[g])

        zi = jnp.zeros((LANE,), jnp.int32)
        rows16s = lax.iota(jnp.int32, LANE) * S
        rows16w = lax.iota(jnp.int32, LANE) * WPAD

        pltpu.make_async_copy(lin_h, lin_t, lsem).wait()

        for gq in range(CB):
            def cate_acc(j, acc):
                idx16 = plsc.load_gather(
                    idx_v, [rows16s + (gq * LANE * S + j)])
                return acc + plsc.load_gather(lin_t, [idx16])
            acc = lax.fori_loop(0, NCAT, cate_acc,
                                jnp.zeros((LANE,), jnp.float32))
            for t in range(NNUM):
                idx16 = plsc.load_gather(
                    idx_v, [rows16s + (gq * LANE * S + NCAT + t)])
                lin16 = plsc.load_gather(lin_t, [idx16])
                wv = plsc.load_gather(
                    w_v, [rows16w + (gq * LANE * WPAD + t)])
                acc = acc + lin16 * wv
            fo_v[pl.ds(gq * LANE, LANE)] = acc

        def ring_body(o, carry):
            for slot in range(NBUF):
                g = o * NBUF + slot
                pltpu.make_async_copy(
                    emb_h.at[idx_v.at[pl.ds(g * RPG * S, RPG * S)]],
                    ebuf.at[slot], esems[slot]).wait()
                for rr in range(RPG):
                    i = g * RPG + rr
                    r0 = rr * S
                    s = [jnp.zeros((LANE,), jnp.float32) for _ in range(4)]
                    ss = [jnp.zeros((LANE,), jnp.float32) for _ in range(4)]
                    for j in range(NCAT):
                        for h in range(2):
                            pair = ebuf[slot, r0 + j, pl.ds(h * 32, 32)]
                            va, vb = plsc.unpack(
                                pair, format=plsc.PackFormat.INTERLEAVED,
                                preferred_element_type=jnp.float32)
                            s[2 * h] = s[2 * h] + va
                            ss[2 * h] = ss[2 * h] + va * va
                            s[2 * h + 1] = s[2 * h + 1] + vb
                            ss[2 * h + 1] = ss[2 * h + 1] + vb * vb
                    for t in range(NNUM):
                        wb = plsc.load_gather(w_v, [zi + (i * WPAD + t)])
                        for h in range(2):
                            pair = ebuf[slot, r0 + NCAT + t,
                                        pl.ds(h * 32, 32)]
                            va, vb = plsc.unpack(
                                pair, format=plsc.PackFormat.INTERLEAVED,
                                preferred_element_type=jnp.float32)
                            va = va * wb
                            vb = vb * wb
                            s[2 * h] = s[2 * h] + va
                            ss[2 * h] = ss[2 * h] + va * va
                            s[2 * h + 1] = s[2 * h + 1] + vb
                            ss[2 * h + 1] = ss[2 * h + 1] + vb * vb
                    for c in range(4):
                        so_v[i, pl.ds(c * LANE, LANE)] = (
                            0.5 * (s[c] * s[c] - ss[c]))

                @pl.when(g + NBUF < NG)
                def _():
                    pltpu.async_copy(
                        emb_h.at[idx_v.at[pl.ds((g + NBUF) * RPG * S,
                                                RPG * S)]],
                        ebuf.at[slot], esems[slot])
            return carry
        lax.fori_loop(0, NG // NBUF, ring_body, 0)

        pltpu.sync_copy(so_v, so_h.at[pl.ds(base, RW)])
        pltpu.sync_copy(fo_v, fo_h.at[pl.ds(base, RW)])

    return k(idx_flat, w_flat, emb_bf, lin_flat)


def _mlp(so, fo, W1, b1, W2, b2, W3t, b3):
    GB = 4
    BB = B // GB

    def body(so_ref, fo_ref, w1_ref, b1_ref, w2_ref, b2_ref, w3_ref, b3_ref,
             out_ref):
        h = jnp.dot(so_ref[...], w1_ref[...],
                    preferred_element_type=jnp.float32)
        h = jnp.maximum(h + b1_ref[...], 0.0)
        h = jnp.dot(h, w2_ref[...], preferred_element_type=jnp.float32)
        h = jnp.maximum(h + b2_ref[...], 0.0)
        z = jnp.sum(h * w3_ref[...], axis=1, keepdims=True) + b3_ref[0, 0]
        out_ref[...] = fo_ref[...] + jax.nn.sigmoid(z)

    return pl.pallas_call(
        body,
        grid=(GB,),
        in_specs=[
            pl.BlockSpec((BB, D), lambda i: (i, 0)),
            pl.BlockSpec((BB, 1), lambda i: (i, 0)),
            pl.BlockSpec((D, 64), lambda i: (0, 0)),
            pl.BlockSpec((1, 64), lambda i: (0, 0)),
            pl.BlockSpec((64, 32), lambda i: (0, 0)),
            pl.BlockSpec((1, 32), lambda i: (0, 0)),
            pl.BlockSpec((1, 32), lambda i: (0, 0)),
            pl.BlockSpec((1, 1), lambda i: (0, 0)),
        ],
        out_specs=pl.BlockSpec((BB, 1), lambda i: (i, 0)),
        out_shape=jax.ShapeDtypeStruct((B, 1), jnp.float32),
    )(so, fo, W1, b1, W2, b2, W3t, b3)


def kernel(category_index, numerical_index, numerical_value, emb_table,
           lin_table, W1, b1, W2, b2, W3, b3):
    ci = category_index.astype(jnp.int32)
    ni = numerical_index.astype(jnp.int32)
    nv = numerical_value.astype(jnp.float32)
    idx = jnp.concatenate([ci, ni, jnp.zeros((B, 1), jnp.int32)], axis=1)
    w_flat = jnp.concatenate(
        [nv, jnp.zeros((B, WPAD - NNUM), jnp.float32)], axis=1).reshape(-1)
    emb_bf = emb_table.astype(jnp.bfloat16)
    lin_flat = lin_table[:, 0]

    so, fo = _sc_pool(idx.reshape(B * S), w_flat, emb_bf, lin_flat)
    out = _mlp(so, fo[:, None], W1[_PERM], b1.reshape(1, 64),
               W2, b2.reshape(1, 32), W3.T, b3.reshape(1, 1))
    return out


# speedup vs baseline: 1.5887x; 1.0075x over previous
import functools

import jax
import jax.numpy as jnp
import numpy as np
from jax import lax
from jax.experimental import pallas as pl
from jax.experimental.pallas import tpu as pltpu
from jax.experimental.pallas import tpu_sc as plsc

B = 4096
D = 64
F = 100000
NCAT = 26
NNUM = 13
S = 40
WPAD = 16
NCORE = 2
NSUB = 16
NW = NCORE * NSUB
RW = B // NW
NBUF = 8
LANE = 16
CB = RW // LANE

_PERM = np.concatenate([
    np.arange(0, 32, 2), np.arange(1, 32, 2),
    np.arange(32, 64, 2), np.arange(33, 64, 2)])


def _sc_pool(idx2d, w2d, emb_bf, lin_flat):
    mesh = plsc.VectorSubcoreMesh(
        core_axis_name="c", subcore_axis_name="s",
        num_cores=NCORE, num_subcores=NSUB)

    @functools.partial(
        pl.kernel,
        out_type=(jax.ShapeDtypeStruct((B, D), jnp.float32),
                  jax.ShapeDtypeStruct((B,), jnp.float32)),
        mesh=mesh,
        scratch_types=[
            pltpu.VMEM((RW, S), jnp.int32),
            pltpu.VMEM((RW, WPAD), jnp.float32),
            pltpu.VMEM((F,), jnp.float32),
            pltpu.VMEM((NBUF, S, D), jnp.bfloat16),
            pltpu.VMEM((RW, D), jnp.float32),
            pltpu.VMEM((RW,), jnp.float32),
            pltpu.SemaphoreType.DMA,
        ] + [pltpu.SemaphoreType.DMA for _ in range(NBUF)],
        compiler_params=pltpu.CompilerParams(
            use_tc_tiling_on_sc=False, needs_layout_passes=False),
    )
    def k(idx_h, w_h, emb_h, lin_h, so_h, fo_h,
          idx_v, w_v, lin_t, ebuf, so_v, fo_v, lsem, *esems):
        wid = lax.axis_index("s") * NCORE + lax.axis_index("c")
        base = wid * RW

        pltpu.async_copy(lin_h, lin_t, lsem)

        pltpu.sync_copy(idx_h.at[pl.ds(base, RW)], idx_v)
        pltpu.sync_copy(w_h.at[pl.ds(base, RW)], w_v)

        for g in range(NBUF):
            pltpu.async_copy(
                emb_h.at[idx_v.at[g]], ebuf.at[g], esems[g])

        zi = jnp.zeros((LANE,), jnp.int32)
        rows16 = lax.iota(jnp.int32, LANE)

        pltpu.make_async_copy(lin_h, lin_t, lsem).wait()

        for gq in range(CB):
            r16 = rows16 + gq * LANE

            def cate_acc(j, acc):
                idx16 = plsc.load_gather(idx_v, [r16, zi + j])
                return acc + plsc.load_gather(lin_t, [idx16])
            acc = lax.fori_loop(0, NCAT, cate_acc,
                                jnp.zeros((LANE,), jnp.float32))
            for t in range(NNUM):
                idx16 = plsc.load_gather(idx_v, [r16, zi + (NCAT + t)])
                lin16 = plsc.load_gather(lin_t, [idx16])
                wv = plsc.load_gather(w_v, [r16, zi + t])
                acc = acc + lin16 * wv
            fo_v[pl.ds(gq * LANE, LANE)] = acc

        def ring_body(o, carry):
            for slot in range(NBUF):
                i = o * NBUF + slot
                pltpu.make_async_copy(
                    emb_h.at[idx_v.at[i]], ebuf.at[slot], esems[slot]).wait()
                s = [jnp.zeros((LANE,), jnp.float32) for _ in range(4)]
                ss = [jnp.zeros((LANE,), jnp.float32) for _ in range(4)]
                for j in range(NCAT):
                    for h in range(2):
                        pair = ebuf[slot, j, pl.ds(h * 32, 32)]
                        va, vb = plsc.unpack(
                            pair, format=plsc.PackFormat.INTERLEAVED,
                            preferred_element_type=jnp.float32)
                        s[2 * h] = s[2 * h] + va
                        ss[2 * h] = ss[2 * h] + va * va
                        s[2 * h + 1] = s[2 * h + 1] + vb
                        ss[2 * h + 1] = ss[2 * h + 1] + vb * vb
                for t in range(NNUM):
                    wb = plsc.load_gather(w_v, [zi + i, zi + t])
                    for h in range(2):
                        pair = ebuf[slot, NCAT + t, pl.ds(h * 32, 32)]
                        va, vb = plsc.unpack(
                            pair, format=plsc.PackFormat.INTERLEAVED,
                            preferred_element_type=jnp.float32)
                        va = va * wb
                        vb = vb * wb
                        s[2 * h] = s[2 * h] + va
                        ss[2 * h] = ss[2 * h] + va * va
                        s[2 * h + 1] = s[2 * h + 1] + vb
                        ss[2 * h + 1] = ss[2 * h + 1] + vb * vb
                for c in range(4):
                    so_v[i, pl.ds(c * LANE, LANE)] = (
                        0.5 * (s[c] * s[c] - ss[c]))

                @pl.when(i + NBUF < RW)
                def _():
                    pltpu.async_copy(
                        emb_h.at[idx_v.at[i + NBUF]], ebuf.at[slot],
                        esems[slot])
            return carry
        lax.fori_loop(0, RW // NBUF, ring_body, 0)

        pltpu.sync_copy(so_v, so_h.at[pl.ds(base, RW)])
        pltpu.sync_copy(fo_v, fo_h.at[pl.ds(base, RW)])

    return k(idx2d, w2d, emb_bf, lin_flat)


def _mlp(so, fo, W1, b1, W2, b2, W3t, b3):
    GB = 4
    BB = B // GB

    def body(so_ref, fo_ref, w1_ref, b1_ref, w2_ref, b2_ref, w3_ref, b3_ref,
             out_ref):
        h = jnp.dot(so_ref[...], w1_ref[...],
                    preferred_element_type=jnp.float32)
        h = jnp.maximum(h + b1_ref[...], 0.0)
        h = jnp.dot(h, w2_ref[...], preferred_element_type=jnp.float32)
        h = jnp.maximum(h + b2_ref[...], 0.0)
        z = jnp.sum(h * w3_ref[...], axis=1, keepdims=True) + b3_ref[0, 0]
        out_ref[...] = fo_ref[...] + jax.nn.sigmoid(z)

    return pl.pallas_call(
        body,
        grid=(GB,),
        in_specs=[
            pl.BlockSpec((BB, D), lambda i: (i, 0)),
            pl.BlockSpec((BB, 1), lambda i: (i, 0)),
            pl.BlockSpec((D, 64), lambda i: (0, 0)),
            pl.BlockSpec((1, 64), lambda i: (0, 0)),
            pl.BlockSpec((64, 32), lambda i: (0, 0)),
            pl.BlockSpec((1, 32), lambda i: (0, 0)),
            pl.BlockSpec((1, 32), lambda i: (0, 0)),
            pl.BlockSpec((1, 1), lambda i: (0, 0)),
        ],
        out_specs=pl.BlockSpec((BB, 1), lambda i: (i, 0)),
        out_shape=jax.ShapeDtypeStruct((B, 1), jnp.float32),
    )(so, fo, W1, b1, W2, b2, W3t, b3)


def kernel(category_index, numerical_index, numerical_value, emb_table,
           lin_table, W1, b1, W2, b2, W3, b3):
    ci = category_index.astype(jnp.int32)
    ni = numerical_index.astype(jnp.int32)
    nv = numerical_value.astype(jnp.float32)
    idx2d = jnp.concatenate([ci, ni, jnp.zeros((B, 1), jnp.int32)], axis=1)
    w2d = jnp.concatenate(
        [nv, jnp.zeros((B, WPAD - NNUM), jnp.float32)], axis=1)
    emb_bf = emb_table.astype(jnp.bfloat16)
    lin_flat = lin_table[:, 0]

    so, fo = _sc_pool(idx2d, w2d, emb_bf, lin_flat)
    out = _mlp(so, fo[:, None], W1[_PERM], b1.reshape(1, 64),
               W2, b2.reshape(1, 32), W3.T, b3.reshape(1, 1))
    return out
